# baseline (speedup 1.0000x reference)
import functools

import jax
import jax.numpy as jnp
from jax import lax
from jax.experimental import pallas as pl
from jax.experimental.pallas import tpu as pltpu
from jax.experimental.pallas import tpu_sc as plsc

_D = 64
_SCALE = 10.0
_GROUP = 128
_NW = 32


def _build(B):
    assert B % (_NW * _GROUP) == 0
    n_groups = B // (_NW * _GROUP)
    b_per_w = n_groups * _GROUP
    mesh = plsc.VectorSubcoreMesh(core_axis_name="c", subcore_axis_name="s")

    @functools.partial(
        pl.kernel,
        mesh=mesh,
        compiler_params=pltpu.CompilerParams(use_tc_tiling_on_sc=False),
        out_type=jax.ShapeDtypeStruct((B, _D), jnp.float32),
        scratch_types=[
            pltpu.VMEM((n_groups, _GROUP), jnp.int32),
            pltpu.VMEM((_GROUP, _D), jnp.float32),
            pltpu.SemaphoreType.DMA,
        ],
    )
    def embed(table_hbm, idx_hbm, out_hbm, idx_v, rows_v, sem):
        wid = lax.axis_index("s") * 2 + lax.axis_index("c")
        base = wid * b_per_w
        pltpu.sync_copy(idx_hbm.at[wid], idx_v)

        def group_body(j, _):
            pltpu.async_copy(table_hbm.at[idx_v.at[j]], rows_v, sem).wait()

            def mul_body(i, _):
                for l in range(_D // 16):
                    sl = pl.ds(l * 16, 16)
                    rows_v[i, sl] = rows_v[i, sl] * _SCALE
                return _

            lax.fori_loop(0, _GROUP, mul_body, None)
            pltpu.sync_copy(rows_v, out_hbm.at[pl.ds(base + j * _GROUP, _GROUP)])
            return _

        lax.fori_loop(0, n_groups, group_body, None)

    return embed


def kernel(x, weight):
    S0, S1 = x.shape
    B = S0 * S1
    n_groups = B // (_NW * _GROUP)
    idx = x.astype(jnp.int32).reshape(_NW, n_groups, _GROUP)
    out = _build(B)(weight, idx)
    return out.reshape(S0, S1, _D)


# speedup vs baseline: 1.1315x; 1.1315x over previous
import functools

import jax
import jax.numpy as jnp
from jax import lax
from jax.experimental import pallas as pl
from jax.experimental.pallas import tpu as pltpu
from jax.experimental.pallas import tpu_sc as plsc

_D = 64
_SCALE = 10.0
_GROUP = 128
_NW = 32


_NBUF = 4


def _build(B):
    assert B % (_NW * _GROUP * _NBUF) == 0
    n_groups = B // (_NW * _GROUP)
    n_rounds = n_groups // _NBUF
    b_per_w = n_groups * _GROUP
    mesh = plsc.VectorSubcoreMesh(core_axis_name="c", subcore_axis_name="s")

    @functools.partial(
        pl.kernel,
        mesh=mesh,
        compiler_params=pltpu.CompilerParams(use_tc_tiling_on_sc=False),
        out_type=jax.ShapeDtypeStruct((B, _D), jnp.float32),
        scratch_types=[
            pltpu.VMEM((n_groups, _GROUP), jnp.int32),
            pltpu.VMEM((_NBUF, _GROUP, _D), jnp.float32),
            pltpu.SemaphoreType.DMA,
        ]
        + [pltpu.SemaphoreType.DMA] * _NBUF
        + [pltpu.SemaphoreType.DMA] * _NBUF,
    )
    def embed(table_hbm, idx_hbm, out_hbm, idx_v, rows_v, isem, *bsems):
        gsem = bsems[:_NBUF]
        ssem = bsems[_NBUF:]
        wid = lax.axis_index("s") * 2 + lax.axis_index("c")
        base = wid * b_per_w
        pltpu.async_copy(idx_hbm.at[wid], idx_v, isem).wait()

        for b in range(_NBUF):
            pltpu.async_copy(table_hbm.at[idx_v.at[b]], rows_v.at[b], gsem[b])

        def round_body(r, _):
            g0 = r * _NBUF
            for b in range(_NBUF):
                pltpu.make_async_copy(
                    table_hbm.at[idx_v.at[0]], rows_v.at[b], gsem[b]
                ).wait()

                def mul_body(i, _, b=b):
                    i2 = i * 2
                    for dr in range(2):
                        for l in range(_D // 16):
                            sl = pl.ds(l * 16, 16)
                            rows_v[b, i2 + dr, sl] = rows_v[b, i2 + dr, sl] * _SCALE
                    return _

                lax.fori_loop(0, _GROUP // 2, mul_body, None)
                pltpu.async_copy(
                    rows_v.at[b],
                    out_hbm.at[pl.ds(base + (g0 + b) * _GROUP, _GROUP)],
                    ssem[b],
                )
            @pl.when(r < n_rounds - 1)
            def _refill():
                for b in range(_NBUF):
                    pltpu.make_async_copy(
                        rows_v.at[b], out_hbm.at[pl.ds(0, _GROUP)], ssem[b]
                    ).wait()
                    pltpu.async_copy(
                        table_hbm.at[idx_v.at[g0 + _NBUF + b]], rows_v.at[b], gsem[b]
                    )

            @pl.when(r == n_rounds - 1)
            def _drain():
                for b in range(_NBUF):
                    pltpu.make_async_copy(
                        rows_v.at[b], out_hbm.at[pl.ds(0, _GROUP)], ssem[b]
                    ).wait()

            return _

        lax.fori_loop(0, n_rounds, round_body, None)

    return embed


def kernel(x, weight):
    S0, S1 = x.shape
    B = S0 * S1
    n_groups = B // (_NW * _GROUP)
    idx = x.astype(jnp.int32).reshape(_NW, n_groups, _GROUP)
    out = _build(B)(weight, idx)
    return out.reshape(S0, S1, _D)
